# baseline (speedup 1.0000x reference)
import jax
import jax.numpy as jnp
from jax.experimental import pallas as pl
from jax.experimental.pallas import tpu as pltpu

_N = 10000
_IN_FT = 256
_OUT_FT = 256
_BM = 400


def _fc_kernel(seq_ref, wt_ref, bfc_ref, sf_ref):
    x = jnp.dot(
        seq_ref[...].astype(jnp.bfloat16),
        wt_ref[...].astype(jnp.bfloat16),
        preferred_element_type=jnp.float32,
    )
    sf_ref[...] = (x + bfc_ref[...]).astype(jnp.bfloat16)


def _agg_kernel(adj_ref, sf_ref, bias_ref, ap_ref, out_ref):
    a = adj_ref[...].astype(jnp.bfloat16)
    acc = jnp.dot(a, sf_ref[...], preferred_element_type=jnp.float32)
    acc = acc + bias_ref[...]
    ap = ap_ref[0, 0]
    out_ref[...] = jnp.where(acc >= 0.0, acc, ap * acc)


def kernel(seq, adj, W_fc, b_fc, bias, a_prelu):
    wt = W_fc.T
    bfc2 = b_fc.reshape(1, _OUT_FT)
    bias2 = bias.reshape(1, _OUT_FT)
    ap2 = a_prelu.reshape(1, 1)

    sf = pl.pallas_call(
        _fc_kernel,
        out_shape=jax.ShapeDtypeStruct((_N, _OUT_FT), jnp.bfloat16),
    )(seq, wt, bfc2)

    out = pl.pallas_call(
        _agg_kernel,
        grid=(_N // _BM,),
        in_specs=[
            pl.BlockSpec((_BM, _N), lambda i: (i, 0)),
            pl.BlockSpec((_N, _OUT_FT), lambda i: (0, 0)),
            pl.BlockSpec((1, _OUT_FT), lambda i: (0, 0)),
            pl.BlockSpec((1, 1), lambda i: (0, 0)),
        ],
        out_specs=pl.BlockSpec((_BM, _OUT_FT), lambda i: (i, 0)),
        out_shape=jax.ShapeDtypeStruct((_N, _OUT_FT), jnp.float32),
        compiler_params=pltpu.CompilerParams(
            dimension_semantics=("arbitrary",),
        ),
    )(adj, sf, bias2, ap2)
    return out


# speedup vs baseline: 1.0298x; 1.0298x over previous
import jax
import jax.numpy as jnp
from jax.experimental import pallas as pl
from jax.experimental.pallas import tpu as pltpu

_N = 10000
_IN_FT = 256
_OUT_FT = 256
_BM = 200


def _gcn_kernel(seq_ref, wt_ref, bfc_ref, adj_ref, bias_ref, ap_ref,
                out_ref, sf_ref):
    @pl.when(pl.program_id(0) == 0)
    def _():
        sf_ref[...] = (
            jnp.dot(seq_ref[...], wt_ref[...],
                    preferred_element_type=jnp.float32)
            + bfc_ref[...]
        )

    acc = jnp.dot(adj_ref[...], sf_ref[...],
                  preferred_element_type=jnp.float32)
    acc = acc + bias_ref[...]
    out_ref[...] = jnp.where(acc >= 0.0, acc, ap_ref[0, 0] * acc)


def kernel(seq, adj, W_fc, b_fc, bias, a_prelu):
    wt = W_fc.T
    bfc2 = b_fc.reshape(1, _OUT_FT)
    bias2 = bias.reshape(1, _OUT_FT)
    ap2 = a_prelu.reshape(1, 1)

    return pl.pallas_call(
        _gcn_kernel,
        grid=(_N // _BM,),
        in_specs=[
            pl.BlockSpec((_N, _IN_FT), lambda i: (0, 0)),
            pl.BlockSpec((_IN_FT, _OUT_FT), lambda i: (0, 0)),
            pl.BlockSpec((1, _OUT_FT), lambda i: (0, 0)),
            pl.BlockSpec((_BM, _N), lambda i: (i, 0)),
            pl.BlockSpec((1, _OUT_FT), lambda i: (0, 0)),
            pl.BlockSpec((1, 1), lambda i: (0, 0)),
        ],
        out_specs=pl.BlockSpec((_BM, _OUT_FT), lambda i: (i, 0)),
        out_shape=jax.ShapeDtypeStruct((_N, _OUT_FT), jnp.float32),
        scratch_shapes=[pltpu.VMEM((_N, _OUT_FT), jnp.float32)],
        compiler_params=pltpu.CompilerParams(
            dimension_semantics=("arbitrary",),
        ),
    )(seq, wt, bfc2, adj, bias2, ap2)


# speedup vs baseline: 1.0420x; 1.0119x over previous
import jax
import jax.numpy as jnp
from jax.experimental import pallas as pl
from jax.experimental.pallas import tpu as pltpu

_N = 10000
_IN_FT = 256
_OUT_FT = 256
_BM = 400


def _gcn_kernel(seq_ref, wt_ref, bfc_ref, adj_ref, bias_ref, ap_ref,
                out_ref, sf_ref):
    @pl.when(pl.program_id(0) == 0)
    def _():
        sf_ref[...] = (
            jnp.dot(seq_ref[...], wt_ref[...],
                    preferred_element_type=jnp.float32)
            + bfc_ref[...]
        )

    acc = jnp.dot(adj_ref[...], sf_ref[...],
                  preferred_element_type=jnp.float32)
    acc = acc + bias_ref[...]
    out_ref[...] = jnp.where(acc >= 0.0, acc, ap_ref[0, 0] * acc)


def kernel(seq, adj, W_fc, b_fc, bias, a_prelu):
    wt = W_fc.T
    bfc2 = b_fc.reshape(1, _OUT_FT)
    bias2 = bias.reshape(1, _OUT_FT)
    ap2 = a_prelu.reshape(1, 1)

    return pl.pallas_call(
        _gcn_kernel,
        grid=(_N // _BM,),
        in_specs=[
            pl.BlockSpec((_N, _IN_FT), lambda i: (0, 0)),
            pl.BlockSpec((_IN_FT, _OUT_FT), lambda i: (0, 0)),
            pl.BlockSpec((1, _OUT_FT), lambda i: (0, 0)),
            pl.BlockSpec((_BM, _N), lambda i: (i, 0)),
            pl.BlockSpec((1, _OUT_FT), lambda i: (0, 0)),
            pl.BlockSpec((1, 1), lambda i: (0, 0)),
        ],
        out_specs=pl.BlockSpec((_BM, _OUT_FT), lambda i: (i, 0)),
        out_shape=jax.ShapeDtypeStruct((_N, _OUT_FT), jnp.float32),
        scratch_shapes=[pltpu.VMEM((_N, _OUT_FT), jnp.float32)],
        compiler_params=pltpu.CompilerParams(
            dimension_semantics=("arbitrary",),
        ),
    )(seq, wt, bfc2, adj, bias2, ap2)
